# baseline (speedup 1.0000x reference)
import functools

import jax
import jax.numpy as jnp
from jax import lax
from jax.experimental import pallas as pl
from jax.experimental.pallas import tpu as pltpu
from jax.experimental.pallas import tpu_sc as plsc

N = 50000
E = 800000
NPAD = 50176
CHUNK = 128
NW = 32
CPW = 200
EPAD = NW * CPW * CHUNK
RPT = NPAD // 32
TPR = NPAD // 16


def _sc_mesh():
    return plsc.VectorSubcoreMesh(core_axis_name="c", subcore_axis_name="s")


def _sc_degree(dst2d):
    @functools.partial(
        pl.kernel,
        out_type=jax.ShapeDtypeStruct((2 * N,), jnp.float32),
        mesh=_sc_mesh(),
        scratch_types=[
            pltpu.VMEM((CPW, CHUNK), jnp.int32),
            pltpu.VMEM((CHUNK,), jnp.float32),
            pltpu.VMEM((448,), jnp.float32),
            pltpu.VMEM_SHARED((NPAD,), jnp.float32),
            pltpu.SemaphoreType.DMA,
        ],
    )
    def k(dst_h, out_h, dst_v, ones_v, zb_v, acc, dsem):
        cid = lax.axis_index("c")
        sid = lax.axis_index("s")
        w = sid * 2 + cid

        @pl.loop(0, 8)
        def _fill(i):
            ones_v[pl.ds(i * 16, 16)] = jnp.full((16,), 1.0, jnp.float32)

        @pl.loop(0, 448 // 16)
        def _zb(i):
            zb_v[pl.ds(i * 16, 16)] = jnp.zeros((16,), jnp.float32)

        @pl.loop(0, TPR // 448)
        def _zero(i):
            pltpu.sync_copy(zb_v, acc.at[pl.ds(sid * TPR + i * 448, 448)])

        plsc.subcore_barrier()

        pltpu.sync_copy(dst_h.at[pl.ds(w * CPW, CPW)], dst_v)

        @pl.loop(0, CPW)
        def _edges(c):
            pltpu.async_copy(ones_v, acc.at[dst_v.at[c]], dsem, add=True)

            @pl.when(c >= 8)
            def _():
                pltpu.make_async_copy(ones_v, acc.at[dst_v.at[0]],
                                      dsem).wait()

        for _ in range(8):
            pltpu.make_async_copy(ones_v, acc.at[dst_v.at[0]], dsem).wait()

        plsc.subcore_barrier()

        for kk in range(7):
            off = sid * TPR + kk * 448
            if kk < 6:
                pltpu.sync_copy(acc.at[pl.ds(off, 448)], zb_v)
                pltpu.sync_copy(zb_v, out_h.at[pl.ds(cid * N + off, 448)])
            else:
                @pl.when(sid < 15)
                def _(off=off):
                    pltpu.sync_copy(acc.at[pl.ds(off, 448)], zb_v)
                    pltpu.sync_copy(zb_v, out_h.at[pl.ds(cid * N + off, 448)])

                @pl.when(sid == 15)
                def _():
                    pltpu.sync_copy(acc.at[pl.ds(49728, 272)],
                                    zb_v.at[pl.ds(0, 272)])
                    pltpu.sync_copy(zb_v.at[pl.ds(0, 272)],
                                    out_h.at[pl.ds(cid * N + 49728, 272)])

    return k(dst2d)


def _sc_aggregate(src2d, dst2d, g):
    @functools.partial(
        pl.kernel,
        out_type=jax.ShapeDtypeStruct((2, N, 16), jnp.float32),
        mesh=_sc_mesh(),
        scratch_types=[
            pltpu.VMEM((CPW, CHUNK), jnp.int32),
            pltpu.VMEM((CPW, CHUNK), jnp.int32),
            pltpu.VMEM((4, CHUNK, 16), jnp.float32),
            pltpu.VMEM((448, 16), jnp.float32),
            pltpu.VMEM_SHARED((NPAD, 16), jnp.float32),
            pltpu.SemaphoreType.DMA,
            pltpu.SemaphoreType.DMA,
            pltpu.SemaphoreType.DMA,
            pltpu.SemaphoreType.DMA,
            pltpu.SemaphoreType.DMA,
            pltpu.SemaphoreType.DMA,
            pltpu.SemaphoreType.DMA,
            pltpu.SemaphoreType.DMA,
        ],
        compiler_params=pltpu.CompilerParams(use_tc_tiling_on_sc=False),
    )
    def k(src_h, dst_h, g_h, out_h, src_v, dst_v, rows_v, zb_v, acc,
          sg0, sg1, sg2, sg3, ss0, ss1, ss2, ss3):
        cid = lax.axis_index("c")
        sid = lax.axis_index("s")
        w = sid * 2 + cid

        @pl.loop(0, 448)
        def _zb(i):
            zb_v[i, :] = jnp.zeros((16,), jnp.float32)

        @pl.loop(0, TPR // 448)
        def _zero(i):
            pltpu.sync_copy(zb_v, acc.at[pl.ds(sid * TPR + i * 448, 448)])

        plsc.subcore_barrier()

        pltpu.sync_copy(src_h.at[pl.ds(w * CPW, CPW)], src_v)
        pltpu.sync_copy(dst_h.at[pl.ds(w * CPW, CPW)], dst_v)

        sems_g = (sg0, sg1, sg2, sg3)
        sems_s = (ss0, ss1, ss2, ss3)
        for b in (0, 1):
            pltpu.async_copy(g_h.at[src_v.at[b]], rows_v.at[b], sems_g[b])

        @pl.loop(0, CPW, step=4)
        def _edges(c):
            for b in range(4):
                j = c + b
                b2 = (b + 2) % 4
                pltpu.make_async_copy(g_h.at[src_v.at[b]], rows_v.at[b],
                                      sems_g[b]).wait()
                pltpu.async_copy(rows_v.at[b], acc.at[dst_v.at[j]],
                                 sems_s[b], add=True)

                @pl.when(j + 2 < CPW)
                def _(b=b, b2=b2, j=j):
                    @pl.when(j >= 2)
                    def _():
                        pltpu.make_async_copy(rows_v.at[b2],
                                              acc.at[dst_v.at[0]],
                                              sems_s[b2]).wait()
                    pltpu.async_copy(g_h.at[src_v.at[j + 2]], rows_v.at[b2],
                                     sems_g[b2])

        for b in range(4):
            pltpu.make_async_copy(rows_v.at[b], acc.at[dst_v.at[0]],
                                  sems_s[b]).wait()

        plsc.subcore_barrier()

        for kk in range(7):
            off = sid * TPR + kk * 448
            if kk < 6:
                pltpu.sync_copy(acc.at[pl.ds(off, 448)], zb_v)
                pltpu.sync_copy(zb_v, out_h.at[cid, pl.ds(off, 448)])
            else:
                @pl.when(sid < 15)
                def _(off=off):
                    pltpu.sync_copy(acc.at[pl.ds(off, 448)], zb_v)
                    pltpu.sync_copy(zb_v, out_h.at[cid, pl.ds(off, 448)])

                @pl.when(sid == 15)
                def _():
                    pltpu.sync_copy(acc.at[pl.ds(49728, 272)],
                                    zb_v.at[pl.ds(0, 272)])
                    pltpu.sync_copy(zb_v.at[pl.ds(0, 272)],
                                    out_h.at[cid, pl.ds(49728, 272)])

    return k(src2d, dst2d, g)


def _sc_filter(src2d, dst2d):
    @functools.partial(
        pl.kernel,
        out_type=(jax.ShapeDtypeStruct((EPAD,), jnp.int32),
                  jax.ShapeDtypeStruct((EPAD,), jnp.int32),
                  jax.ShapeDtypeStruct((NW * 16,), jnp.int32)),
        mesh=_sc_mesh(),
        scratch_types=[
            pltpu.VMEM((CPW, CHUNK), jnp.int32),
            pltpu.VMEM((CPW, CHUNK), jnp.int32),
            pltpu.VMEM((CPW * CHUNK,), jnp.int32),
            pltpu.VMEM((CPW * CHUNK,), jnp.int32),
            pltpu.VMEM((16,), jnp.int32),
        ],
        compiler_params=pltpu.CompilerParams(use_tc_tiling_on_sc=False,
                                             needs_layout_passes=False),
    )
    def k(src_h, dst_h, fsrc_h, fdst_h, cnt_h, src_v, dst_v, fsrc_v, fdst_v,
          cnt_v):
        cid = lax.axis_index("c")
        sid = lax.axis_index("s")
        w = sid * 2 + cid
        pltpu.sync_copy(src_h.at[pl.ds(w * CPW, CPW)], src_v)
        pltpu.sync_copy(dst_h.at[pl.ds(w * CPW, CPW)], dst_v)

        lanes = lax.iota(jnp.int32, 16)

        @pl.loop(0, CPW * CHUNK // 16)
        def _init(i):
            fdst_v[pl.ds(i * 16, 16)] = 2500 + ((lanes + i) & 31)
            fsrc_v[pl.ds(i * 16, 16)] = ((lanes + i * 16) * 53) & 32767

        ONE = jnp.full((16,), 1, jnp.int32)
        ZERO = jnp.full((16,), 0, jnp.int32)

        def body(i, off):
            c = i // 8
            kk = i % 8
            d = dst_v[c, pl.ds(kk * 16, 16)]
            sv = src_v[c, pl.ds(kk * 16, 16)]
            d4 = lax.shift_right_logical(d, 2)
            q = lax.shift_right_logical(d4 * 52429, 18)
            m = ((d & 3) == 0) & ((q * 5) == d4)
            plsc.store_compressed(fdst_v.at[pl.ds(off, 16)], q, mask=m)
            plsc.store_compressed(fsrc_v.at[pl.ds(off, 16)], sv, mask=m)
            return off + jnp.sum(jnp.where(m, ONE, ZERO))

        off = lax.fori_loop(0, CPW * 8, body, jnp.int32(0))
        nch = (off + 127) // 128
        cnt_v[...] = jnp.full((16,), 1, jnp.int32) * nch
        pltpu.sync_copy(cnt_v, cnt_h.at[pl.ds(w * 16, 16)])
        pltpu.sync_copy(fsrc_v, fsrc_h.at[pl.ds(w * CPW * CHUNK, CPW * CHUNK)])
        pltpu.sync_copy(fdst_v, fdst_h.at[pl.ds(w * CPW * CHUNK, CPW * CHUNK)])

    return k(src2d, dst2d)


def _sc_aggregate_filtered(fsrc3, fdst3, cnt, g):
    S = N // 20
    SPAD = 2560

    @functools.partial(
        pl.kernel,
        out_type=jax.ShapeDtypeStruct((2, S, 16), jnp.float32),
        mesh=_sc_mesh(),
        scratch_types=[
            pltpu.VMEM((CPW, CHUNK), jnp.int32),
            pltpu.VMEM((CPW, CHUNK), jnp.int32),
            pltpu.VMEM((CHUNK, 16), jnp.float32),
            pltpu.VMEM((160, 16), jnp.float32),
            pltpu.VMEM((16,), jnp.int32),
            pltpu.VMEM_SHARED((SPAD, 16), jnp.float32),
            pltpu.SemaphoreType.DMA,
        ],
        compiler_params=pltpu.CompilerParams(use_tc_tiling_on_sc=False,
                                             needs_layout_passes=False),
    )
    def k(src_h, dst_h, cnt_h, g_h, out_h, src_v, dst_v, rows_v, zb_v, cnt_v,
          acc, sem0):
        cid = lax.axis_index("c")
        sid = lax.axis_index("s")
        w = sid * 2 + cid

        @pl.loop(0, 160)
        def _zb(i):
            zb_v[i, :] = jnp.zeros((16,), jnp.float32)

        pltpu.sync_copy(zb_v, acc.at[pl.ds(sid * 160, 160)])

        pltpu.sync_copy(cnt_h.at[pl.ds(w * 16, 16)], cnt_v)
        nch = cnt_v[...]

        pltpu.sync_copy(src_h.at[w], src_v)
        pltpu.sync_copy(dst_h.at[w], dst_v)

        plsc.subcore_barrier()

        @pl.loop(0, CPW)
        def _edges(c):
            @pl.when(jnp.all(c < nch))
            def _():
                pltpu.async_copy(g_h.at[src_v.at[c]], rows_v, sem0).wait()
                pltpu.sync_copy(rows_v, acc.at[dst_v.at[c]], add=True)

        plsc.subcore_barrier()

        @pl.when(sid < 15)
        def _():
            pltpu.sync_copy(acc.at[pl.ds(sid * 160, 160)], zb_v)
            pltpu.sync_copy(zb_v, out_h.at[cid, pl.ds(sid * 160, 160)])

        @pl.when(sid == 15)
        def _():
            pltpu.sync_copy(acc.at[pl.ds(2400, 100)], zb_v.at[pl.ds(0, 100)])
            pltpu.sync_copy(zb_v.at[pl.ds(0, 100)],
                            out_h.at[cid, pl.ds(2400, 100)])

    return k(fsrc3, fdst3, cnt, g)


def _tc_scale_matmul(x, d3, w1t):
    blk = 2000

    def body(x_ref, d_ref, w_ref, o_ref):
        d = d_ref[0]
        deg = d[0:1, :] + d[1:2, :] + 1.0
        dis = lax.rsqrt(deg).reshape(blk, 1)
        h = jnp.dot(x_ref[...], w_ref[...], preferred_element_type=jnp.float32)
        o_ref[...] = dis * h

    return pl.pallas_call(
        body,
        grid=(N // blk,),
        in_specs=[
            pl.BlockSpec((blk, 100), lambda i: (i, 0)),
            pl.BlockSpec((1, 2, blk), lambda i: (i, 0, 0)),
            pl.BlockSpec((100, 16), lambda i: (0, 0)),
        ],
        out_specs=pl.BlockSpec((blk, 16), lambda i: (i, 0)),
        out_shape=jax.ShapeDtypeStruct((N, 16), jnp.float32),
    )(x, d3, w1t)


def _tc_relu_rescale(p, g, d3, b1):
    blk = 2000

    def body(p_ref, g_ref, d_ref, b_ref, o_ref):
        d = d_ref[0]
        deg = d[0:1, :] + d[1:2, :] + 1.0
        dis = lax.rsqrt(deg).reshape(blk, 1)
        s = p_ref[0] + p_ref[1] + g_ref[...]
        h1 = jnp.maximum(dis * s + b_ref[...], 0.0)
        o_ref[...] = dis * h1

    return pl.pallas_call(
        body,
        grid=(N // blk,),
        in_specs=[
            pl.BlockSpec((2, blk, 16), lambda i: (0, i, 0)),
            pl.BlockSpec((blk, 16), lambda i: (i, 0)),
            pl.BlockSpec((1, 2, blk), lambda i: (i, 0, 0)),
            pl.BlockSpec((1, 16), lambda i: (0, 0)),
        ],
        out_specs=pl.BlockSpec((blk, 16), lambda i: (i, 0)),
        out_shape=jax.ShapeDtypeStruct((N, 16), jnp.float32),
    )(p, g, d3, b1)


def _tc_tail(p2, g1r, dE, te, ce, pe, w2t, b2, w1mt, b1m, w2mt, b2m,
             w3mt, b3m):
    S = N // 20

    def body(p_ref, g_ref, dE_ref, te_ref, ce_ref, pe_ref, w2_ref,
             b2_ref, m1_ref, c1_ref, m2_ref, c2_ref, m3_ref, c3_ref, o_ref):
        deg = dE_ref[0, :, 0:1] + dE_ref[1, :, 0:1] + 1.0
        dis = lax.rsqrt(deg)
        q = dis * (p_ref[0, :, 0:16] + p_ref[1, :, 0:16] + g_ref[:, 0:16])
        t = jnp.dot(q, w2_ref[...], preferred_element_type=jnp.float32)
        t = t + b2_ref[...]
        m = jnp.max(t, axis=1, keepdims=True)
        ls = t - (m + jnp.log(jnp.sum(jnp.exp(t - m), axis=1, keepdims=True)))
        tev = te_ref[...]
        cev = ce_ref[...]
        pev = pe_ref[...]
        mn = jnp.minimum(jnp.minimum(jnp.min(tev), jnp.min(cev)),
                         jnp.minimum(jnp.min(pev), jnp.min(ls)))
        mx = jnp.maximum(jnp.maximum(jnp.max(tev), jnp.max(cev)),
                         jnp.maximum(jnp.max(pev), jnp.max(ls)))
        scale = 1.0 / (mx - mn)
        m1 = m1_ref[...]
        bsum = (tev * m1[0:1, :] + cev * m1[1:2, :] + pev * m1[2:3, :]
                + jnp.dot(ls, m1[3:5, :], preferred_element_type=jnp.float32))
        rowsum = m1[0:1, :] + m1[1:2, :] + m1[2:3, :] + m1[3:4, :] + m1[4:5, :]
        h = jnp.maximum((bsum - mn * rowsum) * scale + c1_ref[...], 0.0)
        h = jnp.maximum(
            jnp.dot(h, m2_ref[...], preferred_element_type=jnp.float32)
            + c2_ref[...], 0.0)
        z = jnp.dot(h, m3_ref[...], preferred_element_type=jnp.float32) \
            + c3_ref[...]
        o_ref[...] = 1.0 / (1.0 + jnp.exp(-z))

    return pl.pallas_call(
        body,
        grid=(1,),
        in_specs=[
            pl.BlockSpec((2, S, 16), lambda i: (0, 0, 0)),
            pl.BlockSpec((S, 320), lambda i: (0, 0)),
            pl.BlockSpec((2, S, 20), lambda i: (0, 0, 0)),
            pl.BlockSpec((S, 1), lambda i: (0, 0)),
            pl.BlockSpec((S, 1), lambda i: (0, 0)),
            pl.BlockSpec((S, 1), lambda i: (0, 0)),
            pl.BlockSpec((16, 2), lambda i: (0, 0)),
            pl.BlockSpec((1, 2), lambda i: (0, 0)),
            pl.BlockSpec((5, 80), lambda i: (0, 0)),
            pl.BlockSpec((1, 80), lambda i: (0, 0)),
            pl.BlockSpec((80, 10), lambda i: (0, 0)),
            pl.BlockSpec((1, 10), lambda i: (0, 0)),
            pl.BlockSpec((10, 1), lambda i: (0, 0)),
            pl.BlockSpec((1, 1), lambda i: (0, 0)),
        ],
        out_specs=pl.BlockSpec((S, 1), lambda i: (0, 0)),
        out_shape=jax.ShapeDtypeStruct((S, 1), jnp.float32),
    )(p2, g1r, dE, te, ce, pe, w2t, b2, w1mt, b1m, w2mt, b2m, w3mt, b3m)


def kernel(batch1_edge_index, batch1_x, batch2_edge_index, batch2_x,
           transE_input, ComplEx_input, path_input, edit_input, Wh1, bh1,
           Wh2, bh2, Wt1, bt1, Wt2, bt2, Wm1, bm1, Wm2, bm2, Wm3, bm3):
    src = batch1_edge_index[0].astype(jnp.int32)
    dst = batch1_edge_index[1].astype(jnp.int32)
    npad = EPAD - E
    pad_src = (jnp.arange(npad, dtype=jnp.int32) * 37) % N
    pad_dst = N + (jnp.arange(npad, dtype=jnp.int32) % (NPAD - N))
    src2d = jnp.concatenate([src, pad_src]).reshape(EPAD // CHUNK, CHUNK)
    dst2d = jnp.concatenate([dst, pad_dst]).reshape(EPAD // CHUNK, CHUNK)

    deg_p = _sc_degree(dst2d)
    d3 = deg_p.reshape(2, 25, 2000).transpose(1, 0, 2)
    dE = deg_p.reshape(2, N // 20, 20)

    fsrc_f, fdst_f, fcnt = _sc_filter(src2d, dst2d)
    fsrc3 = fsrc_f.reshape(NW, CPW, CHUNK)
    fdst3 = fdst_f.reshape(NW, CPW, CHUNK)

    g = _tc_scale_matmul(batch1_x, d3, Wh1.T)
    p1 = _sc_aggregate(src2d, dst2d, g)
    g1 = _tc_relu_rescale(p1, g, d3, bh1.reshape(1, 16))
    p2 = _sc_aggregate_filtered(fsrc3, fdst3, fcnt, g1)

    S = N // 20
    out = _tc_tail(
        p2,
        g1.reshape(S, 20 * 16),
        dE,
        transE_input.reshape(S, 1),
        ComplEx_input.reshape(S, 1),
        path_input.reshape(S, 1),
        Wh2.T, bh2.reshape(1, 2),
        Wm1.T, bm1.reshape(1, 80),
        Wm2.T, bm2.reshape(1, 10),
        Wm3.T, bm3.reshape(1, 1),
    )
    return out
